# baseline (speedup 1.0000x reference)
import functools

import jax
import jax.numpy as jnp
from jax import lax
from jax.experimental import pallas as pl
from jax.experimental.pallas import tpu as pltpu
from jax.experimental.pallas import tpu_sc as plsc

D = 128
NC = 2
NS = 16
NW = NC * NS
CHUNK = 128
BR = 512


NBUF = 2


@functools.lru_cache(maxsize=None)
def _make_segsum(n_pad, nchunk):
    assert nchunk % 2 == 0 and nchunk >= 4
    rows_per_tile = n_pad // NS
    mesh = plsc.VectorSubcoreMesh(core_axis_name="c", subcore_axis_name="s")

    @functools.partial(
        pl.kernel,
        out_type=jax.ShapeDtypeStruct((NC, n_pad, D), jnp.float32),
        mesh=mesh,
        scratch_types=[
            pltpu.VMEM_SHARED((n_pad, D), jnp.float32),
            [pltpu.VMEM((CHUNK,), jnp.int32) for _ in range(4)],
            [pltpu.VMEM((CHUNK,), jnp.int32) for _ in range(4)],
            [pltpu.VMEM((CHUNK, D), jnp.float32) for _ in range(NBUF)],
            [pltpu.SemaphoreType.DMA for _ in range(4)],
            [pltpu.SemaphoreType.DMA for _ in range(NBUF)],
            [pltpu.SemaphoreType.DMA for _ in range(NBUF)],
        ],
    )
    def segsum(src_hbm, dst_hbm, table_hbm, zeros_hbm, out_hbm,
               acc, sidx, didx, bufs, isems, gsems, ssems):
        c = lax.axis_index("c")
        s = lax.axis_index("s")
        r0 = pl.multiple_of(s * rows_per_tile, 8)
        pltpu.sync_copy(zeros_hbm.at[pl.ds(r0, rows_per_tile)],
                        acc.at[pl.ds(r0, rows_per_tile)])
        plsc.subcore_barrier()

        w = s * NC + c
        src_my = src_hbm.at[w]
        dst_my = dst_hbm.at[w]

        def fire_idx(j, i):
            pltpu.async_copy(src_my.at[j], sidx[i], isems[i])
            pltpu.async_copy(dst_my.at[j], didx[i], isems[i])

        def wait_idx(j, i):
            pltpu.make_async_copy(src_my.at[j], sidx[i], isems[i]).wait()
            pltpu.make_async_copy(dst_my.at[j], didx[i], isems[i]).wait()

        def step(j, b, i, wait_prev_scat, tail_fires):
            bn, i1, i3 = 1 - b, (i + 1) % 4, (i + 3) % 4
            pltpu.make_async_copy(
                table_hbm.at[sidx[i]], bufs[b], gsems[b]).wait()
            pltpu.async_copy(bufs[b], acc.at[didx[i]], ssems[b], add=True)
            if tail_fires >= 2:
                wait_idx(j + 1, i1)
                if wait_prev_scat:
                    pltpu.make_async_copy(
                        bufs[bn], acc.at[didx[i3]], ssems[bn]).wait()
                pltpu.async_copy(table_hbm.at[sidx[i1]], bufs[bn], gsems[bn])
            elif wait_prev_scat:
                pltpu.make_async_copy(
                    bufs[bn], acc.at[didx[i3]], ssems[bn]).wait()
            if tail_fires >= 3:
                fire_idx(j + 3, i3)

        fire_idx(0, 0)
        fire_idx(1, 1)
        fire_idx(2, 2)
        wait_idx(0, 0)
        pltpu.async_copy(table_hbm.at[sidx[0]], bufs[0], gsems[0])
        step(0, 0, 0, False, 3)

        @pl.loop(1, nchunk - 3, step=4)
        def _(g):
            step(g, 1, 1, True, 3)
            step(g + 1, 0, 2, True, 3)
            step(g + 2, 1, 3, True, 3)
            step(g + 3, 0, 0, True, 3)

        step(nchunk - 3, 1, 1, True, 2)
        step(nchunk - 2, 0, 2, True, 2)
        step(nchunk - 1, 1, 3, True, 1)
        pltpu.make_async_copy(bufs[1], acc.at[didx[3]], ssems[1]).wait()

        plsc.subcore_barrier()
        pltpu.sync_copy(acc.at[pl.ds(r0, rows_per_tile)],
                        out_hbm.at[c].at[pl.ds(r0, rows_per_tile)])

    return segsum


def _tc1_body(p_ref, x_ref, wr_ref, wo_ref, b_ref, h_ref):
    agg = p_ref[0] + p_ref[1]
    h = (jnp.dot(agg, wr_ref[...], preferred_element_type=jnp.float32)
         + jnp.dot(x_ref[...], wo_ref[...], preferred_element_type=jnp.float32)
         + b_ref[...])
    h_ref[...] = jnp.maximum(h, 0.0)


def _tc2_body(p_ref, h_ref, wr_ref, wo_ref, b_ref, o_ref):
    agg = p_ref[0] + p_ref[1]
    o = (jnp.dot(agg, wr_ref[...], preferred_element_type=jnp.float32)
         + jnp.dot(h_ref[...], wo_ref[...], preferred_element_type=jnp.float32)
         + b_ref[...])
    o = o - jnp.max(o, axis=1, keepdims=True)
    o_ref[...] = o - jnp.log(jnp.sum(jnp.exp(o), axis=1, keepdims=True))


def _tc_layer(body, partials, dense_in, w_rel, w_root, b, n_pad):
    grid = (n_pad // BR,)
    return pl.pallas_call(
        body,
        grid=grid,
        in_specs=[
            pl.BlockSpec((NC, BR, D), lambda i: (0, i, 0)),
            pl.BlockSpec((BR, D), lambda i: (i, 0)),
            pl.BlockSpec((D, D), lambda i: (0, 0)),
            pl.BlockSpec((D, D), lambda i: (0, 0)),
            pl.BlockSpec((1, D), lambda i: (0, 0)),
        ],
        out_specs=pl.BlockSpec((BR, D), lambda i: (i, 0)),
        out_shape=jax.ShapeDtypeStruct((n_pad, D), jnp.float32),
    )(partials, dense_in, w_rel, w_root, b.reshape(1, D))


def kernel(x, edge_index, W1_rel, W1_root, b1, W2_rel, W2_root, b2):
    n = x.shape[0]
    e = edge_index.shape[1]
    n_pad = -(-(n + 1) // BR) * BR
    nchunk = -(-e // (NW * CHUNK * NBUF)) * NBUF
    e_pad = nchunk * CHUNK * NW

    pad_ar = jnp.arange(e_pad - e, dtype=jnp.int32)
    src = jnp.concatenate(
        [edge_index[0], pad_ar % n]
    ).reshape(NW, nchunk, CHUNK)
    dst = jnp.concatenate(
        [edge_index[1], n + pad_ar % (n_pad - n)]
    ).reshape(NW, nchunk, CHUNK)
    zeros = jnp.zeros((n_pad, D), jnp.float32)
    x_pad = jnp.concatenate([x, jnp.zeros((n_pad - n, D), jnp.float32)], axis=0)

    segsum = _make_segsum(n_pad, nchunk)
    p1 = segsum(src, dst, x_pad, zeros)
    h = _tc_layer(_tc1_body, p1, x_pad, W1_rel, W1_root, b1, n_pad)
    p2 = segsum(src, dst, h, zeros)
    out = _tc_layer(_tc2_body, p2, h, W2_rel, W2_root, b2, n_pad)
    return out[:n]


# speedup vs baseline: 1.1216x; 1.1216x over previous
import functools

import jax
import jax.numpy as jnp
from jax import lax
from jax.experimental import pallas as pl
from jax.experimental.pallas import tpu as pltpu
from jax.experimental.pallas import tpu_sc as plsc

D = 128
NC = 2
NS = 16
NW = NC * NS
CHUNK = 128
BR = 400


@functools.lru_cache(maxsize=None)
def _make_segsum(n_acc, nchunk, n_extra):
    assert nchunk % 2 == 0 and nchunk >= 4 and 0 <= n_extra <= NW
    rows_per_tile = n_acc // NS
    assert rows_per_tile * NS == n_acc and rows_per_tile % 128 == 0
    mesh = plsc.VectorSubcoreMesh(core_axis_name="c", subcore_axis_name="s")

    @functools.partial(
        pl.kernel,
        out_type=jax.ShapeDtypeStruct((NC, n_acc, D), jnp.float32),
        mesh=mesh,
        scratch_types=[
            pltpu.VMEM_SHARED((n_acc, D), jnp.float32),
            [pltpu.VMEM((CHUNK,), jnp.int32) for _ in range(2)],
            [pltpu.VMEM((CHUNK,), jnp.int32) for _ in range(2)],
            [pltpu.VMEM((CHUNK, D), jnp.float32) for _ in range(2)],
            pltpu.VMEM((CHUNK,), jnp.int32),
            pltpu.VMEM((CHUNK,), jnp.int32),
            [pltpu.SemaphoreType.DMA for _ in range(2)],
            [pltpu.SemaphoreType.DMA for _ in range(2)],
            pltpu.SemaphoreType.DMA,
            pltpu.SemaphoreType.DMA,
        ],
    )
    def segsum(edge_hbm, table_hbm, zeros_hbm, out_hbm,
               acc, sidx, didx, bufs, sidx_t, didx_t,
               isems, gsems, isem_t, gsem_t):
        c = lax.axis_index("c")
        s = lax.axis_index("s")
        w = s * NC + c
        base = w * (nchunk * CHUNK)
        src_my = edge_hbm.at[0]
        dst_my = edge_hbm.at[1]

        def fire_idx(j, b):
            off = pl.multiple_of(base + j * CHUNK, CHUNK)
            pltpu.async_copy(src_my.at[pl.ds(off, CHUNK)], sidx[b], isems[b])
            pltpu.async_copy(dst_my.at[pl.ds(off, CHUNK)], didx[b], isems[b])

        def wait_idx(j, b):
            off = pl.multiple_of(base + j * CHUNK, CHUNK)
            pltpu.make_async_copy(
                src_my.at[pl.ds(off, CHUNK)], sidx[b], isems[b]).wait()
            pltpu.make_async_copy(
                dst_my.at[pl.ds(off, CHUNK)], didx[b], isems[b]).wait()

        def fire_gather(b):
            pltpu.async_copy(table_hbm.at[sidx[b]], bufs[b], gsems[b])

        def step(j, b, bn, fire_next_gather, fire_next_idx):
            if fire_next_gather:
                wait_idx(j + 1, bn)
                fire_gather(bn)
            pltpu.make_async_copy(
                table_hbm.at[sidx[b]], bufs[b], gsems[b]).wait()
            pltpu.sync_copy(bufs[b], acc.at[didx[b]], add=True)
            if fire_next_idx:
                fire_idx(j + 2, b)

        off_t = pl.multiple_of((nchunk * NW + w) * CHUNK, CHUNK)

        fire_idx(0, 0)
        fire_idx(1, 1)
        if n_extra:
            @pl.when(w < n_extra)
            def _():
                pltpu.async_copy(src_my.at[pl.ds(off_t, CHUNK)], sidx_t, isem_t)
                pltpu.async_copy(dst_my.at[pl.ds(off_t, CHUNK)], didx_t, isem_t)
        wait_idx(0, 0)
        fire_gather(0)
        if n_extra:
            @pl.when(w < n_extra)
            def _():
                pltpu.make_async_copy(
                    src_my.at[pl.ds(off_t, CHUNK)], sidx_t, isem_t).wait()
                pltpu.make_async_copy(
                    dst_my.at[pl.ds(off_t, CHUNK)], didx_t, isem_t).wait()
                pltpu.async_copy(table_hbm.at[sidx_t], bufs[1], gsem_t)

        r0 = s * rows_per_tile
        pltpu.sync_copy(zeros_hbm.at[pl.ds(r0, rows_per_tile)],
                        acc.at[pl.ds(r0, rows_per_tile)])
        plsc.subcore_barrier()

        if n_extra:
            @pl.when(w < n_extra)
            def _():
                pltpu.make_async_copy(
                    table_hbm.at[sidx_t], bufs[1], gsem_t).wait()
                pltpu.sync_copy(bufs[1], acc.at[didx_t], add=True)

        @pl.loop(0, nchunk - 2, step=2)
        def _(g):
            step(g, 0, 1, True, True)
            step(g + 1, 1, 0, True, True)

        step(nchunk - 2, 0, 1, True, False)
        step(nchunk - 1, 1, 0, False, False)

        plsc.subcore_barrier()
        pltpu.sync_copy(acc.at[pl.ds(r0, rows_per_tile)],
                        out_hbm.at[c].at[pl.ds(r0, rows_per_tile)])

    return segsum


def _tc1_body(p_ref, x_ref, wr_ref, wo_ref, b_ref, h_ref):
    agg = p_ref[0] + p_ref[1]
    h = (jnp.dot(agg, wr_ref[...], preferred_element_type=jnp.float32)
         + jnp.dot(x_ref[...], wo_ref[...], preferred_element_type=jnp.float32)
         + b_ref[...])
    h_ref[...] = jnp.maximum(h, 0.0)


def _tc2_body(p_ref, h_ref, wr_ref, wo_ref, b_ref, o_ref):
    agg = p_ref[0] + p_ref[1]
    o = (jnp.dot(agg, wr_ref[...], preferred_element_type=jnp.float32)
         + jnp.dot(h_ref[...], wo_ref[...], preferred_element_type=jnp.float32)
         + b_ref[...])
    o = o - jnp.max(o, axis=1, keepdims=True)
    o_ref[...] = o - jnp.log(jnp.sum(jnp.exp(o), axis=1, keepdims=True))


def _tc_layer(body, partials, dense_in, w_rel, w_root, b, n):
    grid = (n // BR,)
    return pl.pallas_call(
        body,
        grid=grid,
        in_specs=[
            pl.BlockSpec((NC, BR, D), lambda i: (0, i, 0)),
            pl.BlockSpec((BR, D), lambda i: (i, 0)),
            pl.BlockSpec((D, D), lambda i: (0, 0)),
            pl.BlockSpec((D, D), lambda i: (0, 0)),
            pl.BlockSpec((1, D), lambda i: (0, 0)),
        ],
        out_specs=pl.BlockSpec((BR, D), lambda i: (i, 0)),
        out_shape=jax.ShapeDtypeStruct((n, D), jnp.float32),
    )(partials, dense_in, w_rel, w_root, b.reshape(1, D))


def kernel(x, edge_index, W1_rel, W1_root, b1, W2_rel, W2_root, b2):
    n = x.shape[0]
    e = edge_index.shape[1]
    assert e % CHUNK == 0
    tot = e // CHUNK
    nchunk = (tot // NW) & ~1
    n_extra = tot - nchunk * NW
    assert n_extra <= NW

    n_acc = -(-n // (NS * 128)) * (NS * 128)
    zeros = jnp.zeros((n_acc, D), jnp.float32)
    segsum = _make_segsum(n_acc, nchunk, n_extra)
    p1 = segsum(edge_index, x, zeros)
    h = _tc_layer(_tc1_body, p1, x, W1_rel, W1_root, b1, n)
    p2 = segsum(edge_index, h, zeros)
    return _tc_layer(_tc2_body, p2, h, W2_rel, W2_root, b2, n)


# speedup vs baseline: 1.1828x; 1.0546x over previous
import functools

import jax
import jax.numpy as jnp
from jax import lax
from jax.experimental import pallas as pl
from jax.experimental.pallas import tpu as pltpu
from jax.experimental.pallas import tpu_sc as plsc

D = 128
NC = 2
NS = 16
NW = NC * NS
CHUNK = 128
BR = 1000


@functools.lru_cache(maxsize=None)
def _make_segsum(n_acc, nchunk, n_extra):
    assert nchunk % 2 == 0 and nchunk >= 4 and 0 <= n_extra <= NW
    rows_per_tile = n_acc // NS
    assert rows_per_tile * NS == n_acc and rows_per_tile % 128 == 0
    mesh = plsc.VectorSubcoreMesh(core_axis_name="c", subcore_axis_name="s")

    @functools.partial(
        pl.kernel,
        out_type=jax.ShapeDtypeStruct((NC, n_acc, D), jnp.float32),
        mesh=mesh,
        scratch_types=[
            pltpu.VMEM_SHARED((n_acc, D), jnp.float32),
            [pltpu.VMEM((CHUNK,), jnp.int32) for _ in range(2)],
            [pltpu.VMEM((CHUNK,), jnp.int32) for _ in range(2)],
            [pltpu.VMEM((CHUNK, D), jnp.float32) for _ in range(2)],
            pltpu.VMEM((CHUNK,), jnp.int32),
            pltpu.VMEM((CHUNK,), jnp.int32),
            [pltpu.SemaphoreType.DMA for _ in range(2)],
            [pltpu.SemaphoreType.DMA for _ in range(2)],
            pltpu.SemaphoreType.DMA,
            pltpu.SemaphoreType.DMA,
        ],
    )
    def segsum(edge_hbm, table_hbm, zeros_hbm, out_hbm,
               acc, sidx, didx, bufs, sidx_t, didx_t,
               isems, gsems, isem_t, gsem_t):
        c = lax.axis_index("c")
        s = lax.axis_index("s")
        w = s * NC + c
        base = w * (nchunk * CHUNK)
        src_my = edge_hbm.at[0]
        dst_my = edge_hbm.at[1]

        def fire_idx(j, b):
            off = pl.multiple_of(base + j * CHUNK, CHUNK)
            pltpu.async_copy(src_my.at[pl.ds(off, CHUNK)], sidx[b], isems[b])
            pltpu.async_copy(dst_my.at[pl.ds(off, CHUNK)], didx[b], isems[b])

        def wait_idx(j, b):
            off = pl.multiple_of(base + j * CHUNK, CHUNK)
            pltpu.make_async_copy(
                src_my.at[pl.ds(off, CHUNK)], sidx[b], isems[b]).wait()
            pltpu.make_async_copy(
                dst_my.at[pl.ds(off, CHUNK)], didx[b], isems[b]).wait()

        def fire_gather(b):
            pltpu.async_copy(table_hbm.at[sidx[b]], bufs[b], gsems[b])

        def step(j, b, bn, fire_next_gather, fire_next_idx):
            if fire_next_gather:
                wait_idx(j + 1, bn)
                fire_gather(bn)
            pltpu.make_async_copy(
                table_hbm.at[sidx[b]], bufs[b], gsems[b]).wait()
            pltpu.sync_copy(bufs[b], acc.at[didx[b]], add=True)
            if fire_next_idx:
                fire_idx(j + 2, b)

        off_t = pl.multiple_of((nchunk * NW + w) * CHUNK, CHUNK)

        fire_idx(0, 0)
        fire_idx(1, 1)
        if n_extra:
            @pl.when(w < n_extra)
            def _():
                pltpu.async_copy(src_my.at[pl.ds(off_t, CHUNK)], sidx_t, isem_t)
                pltpu.async_copy(dst_my.at[pl.ds(off_t, CHUNK)], didx_t, isem_t)
        wait_idx(0, 0)
        fire_gather(0)
        if n_extra:
            @pl.when(w < n_extra)
            def _():
                pltpu.make_async_copy(
                    src_my.at[pl.ds(off_t, CHUNK)], sidx_t, isem_t).wait()
                pltpu.make_async_copy(
                    dst_my.at[pl.ds(off_t, CHUNK)], didx_t, isem_t).wait()
                pltpu.async_copy(table_hbm.at[sidx_t], bufs[1], gsem_t)

        r0 = s * rows_per_tile
        pltpu.sync_copy(zeros_hbm.at[pl.ds(r0, rows_per_tile)],
                        acc.at[pl.ds(r0, rows_per_tile)])
        plsc.subcore_barrier()

        if n_extra:
            @pl.when(w < n_extra)
            def _():
                pltpu.make_async_copy(
                    table_hbm.at[sidx_t], bufs[1], gsem_t).wait()
                pltpu.sync_copy(bufs[1], acc.at[didx_t], add=True)

        @pl.loop(0, nchunk - 2, step=2)
        def _(g):
            step(g, 0, 1, True, True)
            step(g + 1, 1, 0, True, True)

        step(nchunk - 2, 0, 1, True, False)
        step(nchunk - 1, 1, 0, False, False)

        plsc.subcore_barrier()
        pltpu.sync_copy(acc.at[pl.ds(r0, rows_per_tile)],
                        out_hbm.at[c].at[pl.ds(r0, rows_per_tile)])

    return segsum


def _tc1_body(p_ref, x_ref, wr_ref, wo_ref, b_ref, h_ref):
    agg = p_ref[0] + p_ref[1]
    h = (jnp.dot(agg, wr_ref[...], preferred_element_type=jnp.float32)
         + jnp.dot(x_ref[...], wo_ref[...], preferred_element_type=jnp.float32)
         + b_ref[...])
    h_ref[...] = jnp.maximum(h, 0.0)


def _tc2_body(p_ref, h_ref, wr_ref, wo_ref, b_ref, o_ref):
    agg = p_ref[0] + p_ref[1]
    o = (jnp.dot(agg, wr_ref[...], preferred_element_type=jnp.float32)
         + jnp.dot(h_ref[...], wo_ref[...], preferred_element_type=jnp.float32)
         + b_ref[...])
    o = o - jnp.max(o, axis=1, keepdims=True)
    o_ref[...] = o - jnp.log(jnp.sum(jnp.exp(o), axis=1, keepdims=True))


def _tc_layer(body, partials, dense_in, w_rel, w_root, b, n):
    grid = (n // BR,)
    return pl.pallas_call(
        body,
        grid=grid,
        in_specs=[
            pl.BlockSpec((NC, BR, D), lambda i: (0, i, 0)),
            pl.BlockSpec((BR, D), lambda i: (i, 0)),
            pl.BlockSpec((D, D), lambda i: (0, 0)),
            pl.BlockSpec((D, D), lambda i: (0, 0)),
            pl.BlockSpec((1, D), lambda i: (0, 0)),
        ],
        out_specs=pl.BlockSpec((BR, D), lambda i: (i, 0)),
        out_shape=jax.ShapeDtypeStruct((n, D), jnp.float32),
    )(partials, dense_in, w_rel, w_root, b.reshape(1, D))


def kernel(x, edge_index, W1_rel, W1_root, b1, W2_rel, W2_root, b2):
    n = x.shape[0]
    e = edge_index.shape[1]
    assert e % CHUNK == 0
    tot = e // CHUNK
    nchunk = (tot // NW) & ~1
    n_extra = tot - nchunk * NW
    assert n_extra <= NW

    n_acc = -(-n // (NS * 128)) * (NS * 128)
    zeros = jnp.zeros((n_acc, D), jnp.float32)
    segsum = _make_segsum(n_acc, nchunk, n_extra)
    p1 = segsum(edge_index, x, zeros)
    h = _tc_layer(_tc1_body, p1, x, W1_rel, W1_root, b1, n)
    p2 = segsum(edge_index, h, zeros)
    return _tc_layer(_tc2_body, p2, h, W2_rel, W2_root, b2, n)


# speedup vs baseline: 1.2058x; 1.0194x over previous
import functools

import jax
import jax.numpy as jnp
from jax import lax
from jax.experimental import pallas as pl
from jax.experimental.pallas import tpu as pltpu
from jax.experimental.pallas import tpu_sc as plsc

D = 128
NC = 2
NS = 16
NW = NC * NS
CHUNK = 128
BR = 2000


@functools.lru_cache(maxsize=None)
def _make_segsum(n_acc, nchunk, n_extra):
    assert nchunk % 2 == 0 and nchunk >= 4 and 0 <= n_extra <= NW
    rows_per_tile = n_acc // NS
    assert rows_per_tile * NS == n_acc and rows_per_tile % 128 == 0
    mesh = plsc.VectorSubcoreMesh(core_axis_name="c", subcore_axis_name="s")

    @functools.partial(
        pl.kernel,
        out_type=jax.ShapeDtypeStruct((NC, n_acc, D), jnp.float32),
        mesh=mesh,
        scratch_types=[
            pltpu.VMEM_SHARED((n_acc, D), jnp.float32),
            [pltpu.VMEM((CHUNK,), jnp.int32) for _ in range(2)],
            [pltpu.VMEM((CHUNK,), jnp.int32) for _ in range(2)],
            [pltpu.VMEM((CHUNK, D), jnp.float32) for _ in range(2)],
            pltpu.VMEM((CHUNK,), jnp.int32),
            pltpu.VMEM((CHUNK,), jnp.int32),
            [pltpu.SemaphoreType.DMA for _ in range(2)],
            [pltpu.SemaphoreType.DMA for _ in range(2)],
            pltpu.SemaphoreType.DMA,
            pltpu.SemaphoreType.DMA,
        ],
    )
    def segsum(edge_hbm, table_hbm, zeros_hbm, out_hbm,
               acc, sidx, didx, bufs, sidx_t, didx_t,
               isems, gsems, isem_t, gsem_t):
        c = lax.axis_index("c")
        s = lax.axis_index("s")
        w = s * NC + c
        base = w * (nchunk * CHUNK)
        src_my = edge_hbm.at[0]
        dst_my = edge_hbm.at[1]

        def fire_idx(j, b):
            off = pl.multiple_of(base + j * CHUNK, CHUNK)
            pltpu.async_copy(src_my.at[pl.ds(off, CHUNK)], sidx[b], isems[b])
            pltpu.async_copy(dst_my.at[pl.ds(off, CHUNK)], didx[b], isems[b])

        def wait_idx(j, b):
            off = pl.multiple_of(base + j * CHUNK, CHUNK)
            pltpu.make_async_copy(
                src_my.at[pl.ds(off, CHUNK)], sidx[b], isems[b]).wait()
            pltpu.make_async_copy(
                dst_my.at[pl.ds(off, CHUNK)], didx[b], isems[b]).wait()

        def fire_gather(b):
            pltpu.async_copy(table_hbm.at[sidx[b]], bufs[b], gsems[b])

        def step(j, b, bn, fire_next_gather, fire_next_idx):
            if fire_next_gather:
                wait_idx(j + 1, bn)
                fire_gather(bn)
            pltpu.make_async_copy(
                table_hbm.at[sidx[b]], bufs[b], gsems[b]).wait()
            pltpu.sync_copy(bufs[b], acc.at[didx[b]], add=True)
            if fire_next_idx:
                fire_idx(j + 2, b)

        off_t = pl.multiple_of((nchunk * NW + w) * CHUNK, CHUNK)

        fire_idx(0, 0)
        fire_idx(1, 1)
        if n_extra:
            @pl.when(w < n_extra)
            def _():
                pltpu.async_copy(src_my.at[pl.ds(off_t, CHUNK)], sidx_t, isem_t)
                pltpu.async_copy(dst_my.at[pl.ds(off_t, CHUNK)], didx_t, isem_t)
        wait_idx(0, 0)
        fire_gather(0)
        if n_extra:
            @pl.when(w < n_extra)
            def _():
                pltpu.make_async_copy(
                    src_my.at[pl.ds(off_t, CHUNK)], sidx_t, isem_t).wait()
                pltpu.make_async_copy(
                    dst_my.at[pl.ds(off_t, CHUNK)], didx_t, isem_t).wait()
                pltpu.async_copy(table_hbm.at[sidx_t], bufs[1], gsem_t)

        r0 = s * rows_per_tile
        pltpu.sync_copy(zeros_hbm.at[pl.ds(r0, rows_per_tile)],
                        acc.at[pl.ds(r0, rows_per_tile)])
        plsc.subcore_barrier()

        if n_extra:
            @pl.when(w < n_extra)
            def _():
                pltpu.make_async_copy(
                    table_hbm.at[sidx_t], bufs[1], gsem_t).wait()
                pltpu.sync_copy(bufs[1], acc.at[didx_t], add=True)

        @pl.loop(0, nchunk - 2, step=2)
        def _(g):
            step(g, 0, 1, True, True)
            step(g + 1, 1, 0, True, True)

        step(nchunk - 2, 0, 1, True, False)
        step(nchunk - 1, 1, 0, False, False)

        plsc.subcore_barrier()
        pltpu.sync_copy(acc.at[pl.ds(r0, rows_per_tile)],
                        out_hbm.at[c].at[pl.ds(r0, rows_per_tile)])

    return segsum


def _tc1_body(p_ref, x_ref, wr_ref, wo_ref, b_ref, h_ref):
    agg = p_ref[0] + p_ref[1]
    h = (jnp.dot(agg, wr_ref[...], preferred_element_type=jnp.float32)
         + jnp.dot(x_ref[...], wo_ref[...], preferred_element_type=jnp.float32)
         + b_ref[...])
    h_ref[...] = jnp.maximum(h, 0.0)


def _tc2_body(p_ref, h_ref, wr_ref, wo_ref, b_ref, o_ref):
    agg = p_ref[0] + p_ref[1]
    o = (jnp.dot(agg, wr_ref[...], preferred_element_type=jnp.float32)
         + jnp.dot(h_ref[...], wo_ref[...], preferred_element_type=jnp.float32)
         + b_ref[...])
    o = o - jnp.max(o, axis=1, keepdims=True)
    o_ref[...] = o - jnp.log(jnp.sum(jnp.exp(o), axis=1, keepdims=True))


def _tc_layer(body, partials, dense_in, w_rel, w_root, b, n):
    grid = (n // BR,)
    return pl.pallas_call(
        body,
        grid=grid,
        in_specs=[
            pl.BlockSpec((NC, BR, D), lambda i: (0, i, 0)),
            pl.BlockSpec((BR, D), lambda i: (i, 0)),
            pl.BlockSpec((D, D), lambda i: (0, 0)),
            pl.BlockSpec((D, D), lambda i: (0, 0)),
            pl.BlockSpec((1, D), lambda i: (0, 0)),
        ],
        out_specs=pl.BlockSpec((BR, D), lambda i: (i, 0)),
        out_shape=jax.ShapeDtypeStruct((n, D), jnp.float32),
    )(partials, dense_in, w_rel, w_root, b.reshape(1, D))


def kernel(x, edge_index, W1_rel, W1_root, b1, W2_rel, W2_root, b2):
    n = x.shape[0]
    e = edge_index.shape[1]
    assert e % CHUNK == 0
    tot = e // CHUNK
    nchunk = (tot // NW) & ~1
    n_extra = tot - nchunk * NW
    assert n_extra <= NW

    n_acc = -(-n // (NS * 128)) * (NS * 128)
    zeros = jnp.zeros((n_acc, D), jnp.float32)
    segsum = _make_segsum(n_acc, nchunk, n_extra)
    p1 = segsum(edge_index, x, zeros)
    h = _tc_layer(_tc1_body, p1, x, W1_rel, W1_root, b1, n)
    p2 = segsum(edge_index, h, zeros)
    return _tc_layer(_tc2_body, p2, h, W2_rel, W2_root, b2, n)


# speedup vs baseline: 1.2195x; 1.0113x over previous
import functools

import jax
import jax.numpy as jnp
from jax import lax
from jax.experimental import pallas as pl
from jax.experimental.pallas import tpu as pltpu
from jax.experimental.pallas import tpu_sc as plsc

D = 128
NC = 2
NS = 16
NW = NC * NS
CHUNK = 128
BR = 5000


@functools.lru_cache(maxsize=None)
def _make_segsum(n_acc, nchunk, n_extra):
    assert nchunk % 2 == 0 and nchunk >= 4 and 0 <= n_extra <= NW
    rows_per_tile = n_acc // NS
    assert rows_per_tile * NS == n_acc and rows_per_tile % 128 == 0
    mesh = plsc.VectorSubcoreMesh(core_axis_name="c", subcore_axis_name="s")

    @functools.partial(
        pl.kernel,
        out_type=jax.ShapeDtypeStruct((NC, n_acc, D), jnp.float32),
        mesh=mesh,
        scratch_types=[
            pltpu.VMEM_SHARED((n_acc, D), jnp.float32),
            [pltpu.VMEM((CHUNK,), jnp.int32) for _ in range(2)],
            [pltpu.VMEM((CHUNK,), jnp.int32) for _ in range(2)],
            [pltpu.VMEM((CHUNK, D), jnp.float32) for _ in range(2)],
            pltpu.VMEM((CHUNK,), jnp.int32),
            pltpu.VMEM((CHUNK,), jnp.int32),
            [pltpu.SemaphoreType.DMA for _ in range(2)],
            [pltpu.SemaphoreType.DMA for _ in range(2)],
            pltpu.SemaphoreType.DMA,
            pltpu.SemaphoreType.DMA,
        ],
    )
    def segsum(edge_hbm, table_hbm, zeros_hbm, out_hbm,
               acc, sidx, didx, bufs, sidx_t, didx_t,
               isems, gsems, isem_t, gsem_t):
        c = lax.axis_index("c")
        s = lax.axis_index("s")
        w = s * NC + c
        base = w * (nchunk * CHUNK)
        src_my = edge_hbm.at[0]
        dst_my = edge_hbm.at[1]

        def fire_idx(j, b):
            off = pl.multiple_of(base + j * CHUNK, CHUNK)
            pltpu.async_copy(src_my.at[pl.ds(off, CHUNK)], sidx[b], isems[b])
            pltpu.async_copy(dst_my.at[pl.ds(off, CHUNK)], didx[b], isems[b])

        def wait_idx(j, b):
            off = pl.multiple_of(base + j * CHUNK, CHUNK)
            pltpu.make_async_copy(
                src_my.at[pl.ds(off, CHUNK)], sidx[b], isems[b]).wait()
            pltpu.make_async_copy(
                dst_my.at[pl.ds(off, CHUNK)], didx[b], isems[b]).wait()

        def fire_gather(b):
            pltpu.async_copy(table_hbm.at[sidx[b]], bufs[b], gsems[b])

        def step(j, b, bn, fire_next_gather, fire_next_idx):
            if fire_next_gather:
                wait_idx(j + 1, bn)
                fire_gather(bn)
            pltpu.make_async_copy(
                table_hbm.at[sidx[b]], bufs[b], gsems[b]).wait()
            pltpu.sync_copy(bufs[b], acc.at[didx[b]], add=True)
            if fire_next_idx:
                fire_idx(j + 2, b)

        off_t = pl.multiple_of((nchunk * NW + w) * CHUNK, CHUNK)

        fire_idx(0, 0)
        fire_idx(1, 1)
        if n_extra:
            @pl.when(w < n_extra)
            def _():
                pltpu.async_copy(src_my.at[pl.ds(off_t, CHUNK)], sidx_t, isem_t)
                pltpu.async_copy(dst_my.at[pl.ds(off_t, CHUNK)], didx_t, isem_t)
        wait_idx(0, 0)
        fire_gather(0)
        if n_extra:
            @pl.when(w < n_extra)
            def _():
                pltpu.make_async_copy(
                    src_my.at[pl.ds(off_t, CHUNK)], sidx_t, isem_t).wait()
                pltpu.make_async_copy(
                    dst_my.at[pl.ds(off_t, CHUNK)], didx_t, isem_t).wait()
                pltpu.async_copy(table_hbm.at[sidx_t], bufs[1], gsem_t)

        r0 = s * rows_per_tile
        pltpu.sync_copy(zeros_hbm.at[pl.ds(r0, rows_per_tile)],
                        acc.at[pl.ds(r0, rows_per_tile)])
        plsc.subcore_barrier()

        if n_extra:
            @pl.when(w < n_extra)
            def _():
                pltpu.make_async_copy(
                    table_hbm.at[sidx_t], bufs[1], gsem_t).wait()
                pltpu.sync_copy(bufs[1], acc.at[didx_t], add=True)

        @pl.loop(0, nchunk - 2, step=2)
        def _(g):
            step(g, 0, 1, True, True)
            step(g + 1, 1, 0, True, True)

        step(nchunk - 2, 0, 1, True, False)
        step(nchunk - 1, 1, 0, False, False)

        plsc.subcore_barrier()
        pltpu.sync_copy(acc.at[pl.ds(r0, rows_per_tile)],
                        out_hbm.at[c].at[pl.ds(r0, rows_per_tile)])

    return segsum


def _tc1_body(p_ref, x_ref, wr_ref, wo_ref, b_ref, h_ref):
    agg = p_ref[0] + p_ref[1]
    h = (jnp.dot(agg, wr_ref[...], preferred_element_type=jnp.float32)
         + jnp.dot(x_ref[...], wo_ref[...], preferred_element_type=jnp.float32)
         + b_ref[...])
    h_ref[...] = jnp.maximum(h, 0.0)


def _tc2_body(p_ref, h_ref, wr_ref, wo_ref, b_ref, o_ref):
    agg = p_ref[0] + p_ref[1]
    o = (jnp.dot(agg, wr_ref[...], preferred_element_type=jnp.float32)
         + jnp.dot(h_ref[...], wo_ref[...], preferred_element_type=jnp.float32)
         + b_ref[...])
    o = o - jnp.max(o, axis=1, keepdims=True)
    o_ref[...] = o - jnp.log(jnp.sum(jnp.exp(o), axis=1, keepdims=True))


def _tc_layer(body, partials, dense_in, w_rel, w_root, b, n):
    grid = (n // BR,)
    return pl.pallas_call(
        body,
        grid=grid,
        in_specs=[
            pl.BlockSpec((NC, BR, D), lambda i: (0, i, 0)),
            pl.BlockSpec((BR, D), lambda i: (i, 0)),
            pl.BlockSpec((D, D), lambda i: (0, 0)),
            pl.BlockSpec((D, D), lambda i: (0, 0)),
            pl.BlockSpec((1, D), lambda i: (0, 0)),
        ],
        out_specs=pl.BlockSpec((BR, D), lambda i: (i, 0)),
        out_shape=jax.ShapeDtypeStruct((n, D), jnp.float32),
    )(partials, dense_in, w_rel, w_root, b.reshape(1, D))


def kernel(x, edge_index, W1_rel, W1_root, b1, W2_rel, W2_root, b2):
    n = x.shape[0]
    e = edge_index.shape[1]
    assert e % CHUNK == 0
    tot = e // CHUNK
    nchunk = (tot // NW) & ~1
    n_extra = tot - nchunk * NW
    assert n_extra <= NW

    n_acc = -(-n // (NS * 128)) * (NS * 128)
    zeros = jnp.zeros((n_acc, D), jnp.float32)
    segsum = _make_segsum(n_acc, nchunk, n_extra)
    p1 = segsum(edge_index, x, zeros)
    h = _tc_layer(_tc1_body, p1, x, W1_rel, W1_root, b1, n)
    p2 = segsum(edge_index, h, zeros)
    return _tc_layer(_tc2_body, p2, h, W2_rel, W2_root, b2, n)
